# baseline (speedup 1.0000x reference)
import functools

import jax
import jax.numpy as jnp
from jax import lax
from jax.experimental import pallas as pl
from jax.experimental.pallas import tpu as pltpu
from jax.experimental.pallas import tpu_sc as plsc

B = 8
V = 10000
F = 256
NNEW = 20000
NC, NS, L = 2, 16, 16
NW = NC * NS
PER_W = (B * NNEW) // NW
WPB = NW // B
N_PER_W = NNEW // WPB
COPY_W = 2504
COPY_LAST = V - COPY_W
CHUNK = 128
NCH = -(-PER_W // CHUNK)
LAST_OFF = PER_W - CHUNK
VOUT = V + NNEW


def _sc_kernel(xf, p1, p2, out, idx1_v, idx2_v, buf1, buf2, obuf,
               sem1, sem2, csem):
    w = lax.axis_index("s") * NC + lax.axis_index("c")
    b = w // WPB
    part = w % WPB
    boff = (b * V).astype(jnp.int32)

    coff = jnp.minimum(part * COPY_W, COPY_LAST)
    copy = pltpu.make_async_copy(
        xf.at[pl.ds(b * V + coff, COPY_W)],
        out.at[pl.ds(b * VOUT + coff, COPY_W)], csem)
    copy.start()

    n0 = part * N_PER_W
    orow0 = b * VOUT + V + n0

    def chunk_body(g, carry):
        off = jnp.minimum(g * CHUNK, LAST_OFF)
        pltpu.sync_copy(p1.at[pl.ds(n0 + off, CHUNK)], idx1_v)
        pltpu.sync_copy(p2.at[pl.ds(n0 + off, CHUNK)], idx2_v)
        for j in range(CHUNK // L):
            sl = pl.ds(j * L, L)
            idx1_v[sl] = idx1_v[sl] + boff
            idx2_v[sl] = idx2_v[sl] + boff
        g1 = pltpu.make_async_copy(xf.at[idx1_v], buf1, sem1)
        g2 = pltpu.make_async_copy(xf.at[idx2_v], buf2, sem2)
        g1.start()
        g2.start()
        g1.wait()
        g2.wait()

        def row_body(r, c):
            for j in range(F // L):
                sl = pl.ds(j * L, L)
                obuf[r, sl] = (buf1[r, sl] + buf2[r, sl]) * 0.5
            return c
        lax.fori_loop(0, CHUNK, row_body, 0, unroll=False)
        pltpu.sync_copy(obuf, out.at[pl.ds(orow0 + off, CHUNK)])
        return carry

    lax.fori_loop(0, NCH, chunk_body, 0, unroll=False)
    copy.wait()


@jax.jit
def _unpool(xf, p1, p2):
    mesh = plsc.VectorSubcoreMesh(core_axis_name="c", subcore_axis_name="s")
    f = pl.kernel(
        _sc_kernel,
        out_type=jax.ShapeDtypeStruct((B * VOUT, F), jnp.float32),
        mesh=mesh,
        scratch_types=[
            pltpu.VMEM((CHUNK,), jnp.int32),
            pltpu.VMEM((CHUNK,), jnp.int32),
            pltpu.VMEM((CHUNK, F), jnp.float32),
            pltpu.VMEM((CHUNK, F), jnp.float32),
            pltpu.VMEM((CHUNK, F), jnp.float32),
            pltpu.SemaphoreType.DMA,
            pltpu.SemaphoreType.DMA,
            pltpu.SemaphoreType.DMA,
        ],
    )
    return f(xf, p1, p2)


def kernel(x, pool_x1, pool_x2):
    xf = x.reshape(B * V, F)
    out = _unpool(xf, pool_x1.astype(jnp.int32), pool_x2.astype(jnp.int32))
    return out.reshape(B, VOUT, F)
